# Initial kernel scaffold; baseline (speedup 1.0000x reference)
import functools

import jax
import jax.numpy as jnp
from jax import lax
from jax.experimental import pallas as pl
from jax.experimental.pallas import tpu as pltpu
from jax.experimental.pallas import tpu_sc as plsc

N = 10000
E = 320000
D = 128

CHUNK = 128
NUM_CHUNKS = E // CHUNK
NC = 2
NS = 16
NW = NC * NS
ROWS_PER_TILE = N // NS
CHUNKS_PER_W = NUM_CHUNKS // NW
EXTRA = NUM_CHUNKS - CHUNKS_PER_W * NW


def _sc_aggregate(x, src, dst, zeros):
    mesh = plsc.VectorSubcoreMesh(core_axis_name="c", subcore_axis_name="s")

    @functools.partial(
        pl.kernel,
        mesh=mesh,
        out_type=jax.ShapeDtypeStruct((NC, N, D), jnp.float32),
        scratch_types=[
            pltpu.VMEM((CHUNK,), jnp.int32),
            pltpu.VMEM((1, CHUNK), jnp.int32),
            pltpu.VMEM((CHUNK, D), jnp.float32),
            pltpu.VMEM_SHARED((N, D), jnp.float32),
            pltpu.SemaphoreType.DMA,
        ],
    )
    def agg_kernel(x_hbm, src_hbm, dst_hbm, zero_hbm, out_hbm,
                   src_v, dst_v, rows_v, acc, sem):
        c = lax.axis_index("c")
        s = lax.axis_index("s")
        w = c * NS + s
        row0 = s * ROWS_PER_TILE

        pltpu.sync_copy(zero_hbm.at[pl.ds(row0, ROWS_PER_TILE)],
                        acc.at[pl.ds(row0, ROWS_PER_TILE)])
        plsc.subcore_barrier()

        nch = CHUNKS_PER_W + jnp.where(w < EXTRA, 1, 0)
        base = CHUNKS_PER_W * w + jnp.minimum(w, EXTRA)

        def body(j, carry):
            @pl.when(j < nch)
            def _():
                off = (base + j) * CHUNK
                pltpu.sync_copy(src_hbm.at[pl.ds(off, CHUNK)], src_v)
                pltpu.sync_copy(dst_hbm.at[pl.ds(off, CHUNK)], dst_v.at[0])
                pltpu.async_copy(x_hbm.at[src_v], rows_v, sem).wait()
                pltpu.sync_copy(rows_v, acc.at[dst_v.at[0]], add=True)
            return carry

        lax.fori_loop(0, CHUNKS_PER_W + 1, body, 0)
        plsc.subcore_barrier()

        pltpu.sync_copy(acc.at[pl.ds(row0, ROWS_PER_TILE)],
                        out_hbm.at[c, pl.ds(row0, ROWS_PER_TILE)])

    return agg_kernel(x, src, dst, zeros)


BLK = 1000


def _mlp_body(eps_ref, x_ref, p_ref, w1_ref, b1_ref, w2_ref, b2_ref, o_ref):
    agg = p_ref[0] + p_ref[1]
    out = (1.0 + eps_ref[...]) * x_ref[...] + agg
    h = jnp.dot(out, w1_ref[...], preferred_element_type=jnp.float32)
    h = jnp.maximum(h + b1_ref[...], 0.0)
    o_ref[...] = (
        jnp.dot(h, w2_ref[...], preferred_element_type=jnp.float32)
        + b2_ref[...]
    )


def _mlp(x, partials, eps, W1, b1, W2, b2):
    eps2 = eps.reshape(1, 1).astype(jnp.float32)
    return pl.pallas_call(
        _mlp_body,
        grid=(N // BLK,),
        in_specs=[
            pl.BlockSpec((1, 1), lambda i: (0, 0)),
            pl.BlockSpec((BLK, D), lambda i: (i, 0)),
            pl.BlockSpec((NC, BLK, D), lambda i: (0, i, 0)),
            pl.BlockSpec((D, D), lambda i: (0, 0)),
            pl.BlockSpec((1, D), lambda i: (0, 0)),
            pl.BlockSpec((D, D), lambda i: (0, 0)),
            pl.BlockSpec((1, D), lambda i: (0, 0)),
        ],
        out_specs=pl.BlockSpec((BLK, D), lambda i: (i, 0)),
        out_shape=jax.ShapeDtypeStruct((N, D), jnp.float32),
    )(eps2, x, partials, W1, b1.reshape(1, D), W2, b2.reshape(1, D))


@jax.jit
def kernel(x, edge_idx, eps, W1, b1, W2, b2):
    ei = edge_idx.astype(jnp.int32)
    src = ei[0]
    dst = ei[1]
    zeros = jnp.zeros((N, D), jnp.float32)
    partials = _sc_aggregate(x, src, dst, zeros)
    return _mlp(x, partials, eps, W1, b1, W2, b2)


# speedup vs baseline: 6.0442x; 6.0442x over previous
import functools

import jax
import jax.numpy as jnp
from jax import lax
from jax.experimental import pallas as pl
from jax.experimental.pallas import tpu as pltpu
from jax.experimental.pallas import tpu_sc as plsc

N = 10000
E = 320000
D = 128

CHUNK = 128
NUM_CHUNKS = E // CHUNK
NC = 2
NS = 16
NW = NC * NS
ROWS_PER_TILE = 624
ROWS_REM = N - NS * ROWS_PER_TILE
CHUNKS_PER_W = NUM_CHUNKS // NW
EXTRA = NUM_CHUNKS - CHUNKS_PER_W * NW


def _sc_aggregate(x, src, dst, zeros):
    mesh = plsc.VectorSubcoreMesh(core_axis_name="c", subcore_axis_name="s")

    @functools.partial(
        pl.kernel,
        mesh=mesh,
        out_type=jax.ShapeDtypeStruct((NC, N, D), jnp.float32),
        scratch_types=[
            pltpu.VMEM((CHUNK,), jnp.int32),
            pltpu.VMEM((1, CHUNK), jnp.int32),
            pltpu.VMEM((CHUNK, D), jnp.float32),
            pltpu.VMEM_SHARED((N, D), jnp.float32),
            pltpu.SemaphoreType.DMA,
        ],
    )
    def agg_kernel(x_hbm, src_hbm, dst_hbm, zero_hbm, out_hbm,
                   src_v, dst_v, rows_v, acc, sem):
        c = lax.axis_index("c")
        s = lax.axis_index("s")
        w = c * NS + s
        row0 = s * ROWS_PER_TILE

        pltpu.sync_copy(zero_hbm.at[pl.ds(row0, ROWS_PER_TILE)],
                        acc.at[pl.ds(row0, ROWS_PER_TILE)])

        @pl.when(s == 0)
        def _():
            pltpu.sync_copy(zero_hbm.at[pl.ds(NS * ROWS_PER_TILE, ROWS_REM)],
                            acc.at[pl.ds(NS * ROWS_PER_TILE, ROWS_REM)])

        plsc.subcore_barrier()

        nch = CHUNKS_PER_W + jnp.where(w < EXTRA, 1, 0)
        base = CHUNKS_PER_W * w + jnp.minimum(w, EXTRA)

        def body(j, carry):
            @pl.when(j < nch)
            def _():
                off = (base + j) * CHUNK
                pltpu.sync_copy(src_hbm.at[pl.ds(off, CHUNK)], src_v)
                pltpu.sync_copy(dst_hbm.at[pl.ds(off, CHUNK)], dst_v.at[0])
                pltpu.async_copy(x_hbm.at[src_v], rows_v, sem).wait()
                pltpu.sync_copy(rows_v, acc.at[dst_v.at[0]], add=True)
            return carry

        lax.fori_loop(0, CHUNKS_PER_W + 1, body, 0)
        plsc.subcore_barrier()

        pltpu.sync_copy(acc.at[pl.ds(row0, ROWS_PER_TILE)],
                        out_hbm.at[c, pl.ds(row0, ROWS_PER_TILE)])

        @pl.when(s == 0)
        def _():
            pltpu.sync_copy(acc.at[pl.ds(NS * ROWS_PER_TILE, ROWS_REM)],
                            out_hbm.at[c, pl.ds(NS * ROWS_PER_TILE, ROWS_REM)])

    return agg_kernel(x, src, dst, zeros)


BLK = 1000


def _mlp_body(eps_ref, x_ref, p_ref, w1_ref, b1_ref, w2_ref, b2_ref, o_ref):
    agg = p_ref[0] + p_ref[1]
    out = (1.0 + eps_ref[...]) * x_ref[...] + agg
    h = jnp.dot(out, w1_ref[...], preferred_element_type=jnp.float32)
    h = jnp.maximum(h + b1_ref[...], 0.0)
    o_ref[...] = (
        jnp.dot(h, w2_ref[...], preferred_element_type=jnp.float32)
        + b2_ref[...]
    )


def _mlp(x, partials, eps, W1, b1, W2, b2):
    eps2 = eps.reshape(1, 1).astype(jnp.float32)
    return pl.pallas_call(
        _mlp_body,
        grid=(N // BLK,),
        in_specs=[
            pl.BlockSpec((1, 1), lambda i: (0, 0)),
            pl.BlockSpec((BLK, D), lambda i: (i, 0)),
            pl.BlockSpec((NC, BLK, D), lambda i: (0, i, 0)),
            pl.BlockSpec((D, D), lambda i: (0, 0)),
            pl.BlockSpec((1, D), lambda i: (0, 0)),
            pl.BlockSpec((D, D), lambda i: (0, 0)),
            pl.BlockSpec((1, D), lambda i: (0, 0)),
        ],
        out_specs=pl.BlockSpec((BLK, D), lambda i: (i, 0)),
        out_shape=jax.ShapeDtypeStruct((N, D), jnp.float32),
    )(eps2, x, partials, W1, b1.reshape(1, D), W2, b2.reshape(1, D))


@jax.jit
def kernel(x, edge_idx, eps, W1, b1, W2, b2):
    ei = edge_idx.astype(jnp.int32)
    src = ei[0]
    dst = ei[1]
    zeros = jnp.zeros((N, D), jnp.float32)
    partials = _sc_aggregate(x, src, dst, zeros)
    return _mlp(x, partials, eps, W1, b1, W2, b2)
